# baseline (speedup 1.0000x reference)
import numpy as np
import jax
import jax.numpy as jnp
from jax.experimental import pallas as pl
from jax.experimental.pallas import tpu as pltpu

_RATIO = 0.6
_RATE = 0.5


def _band_vectors(B, H, W):
    rng = np.random.default_rng(0)
    mask_size = int(max(H, W) * 2)
    lo = int(min(H * 0.5, W * 0.3))
    hi = int(max(H * 0.5, W * 0.3)) + 1
    top = (mask_size - H) // 2
    left = (mask_size - W) // 2
    rows = np.zeros((B, H), np.float32)
    cols = np.zeros((B, W), np.float32)
    for b in range(B):
        gridblock = int(rng.integers(lo, hi))
        if _RATIO == 1:
            length = int(rng.integers(1, gridblock + 1))
        else:
            length = int(min(max(int(gridblock * _RATIO + 0.5), 1), gridblock - 1))
        ind = []
        for _ in range(2):
            start_w = int(rng.integers(0, gridblock + 1))
            v = np.zeros(mask_size, np.float32)
            for i in range(mask_size // gridblock):
                s = gridblock * i + start_w
                e = min(s + length, mask_size)
                if e > s:
                    v[s:e] = 1.0
            ind.append(v)
        rate_cond = rng.random() < _RATE
        if rate_cond:
            rows[b] = ind[0][top:top + H]
            cols[b] = ind[1][left:left + W]
        else:
            rows[b] = 1.0
            cols[b] = 1.0
    return rows, cols


def _body(img_ref, row_ref, col_ref, out_ref):
    r = row_ref[:, 0, :]
    c = col_ref[:, 0, :]
    m = jnp.maximum(r[:, :, None], c[:, None, :])
    out_ref[...] = img_ref[...] * m[:, None]


def kernel(images):
    B, H, W, C = images.shape
    rows_np, cols_np = _band_vectors(B, H, W)
    rows = jnp.asarray(rows_np.reshape(B, 1, H))
    cols = jnp.asarray(cols_np.reshape(B, 1, W))
    img_t = jnp.transpose(images, (0, 3, 1, 2))
    N = 8
    HB = H // 4
    out_t = pl.pallas_call(
        _body,
        grid=(B // N, H // HB),
        in_specs=[
            pl.BlockSpec((N, C, HB, W), lambda b, h: (b, 0, h, 0)),
            pl.BlockSpec((N, 1, HB), lambda b, h: (b, 0, h)),
            pl.BlockSpec((N, 1, W), lambda b, h: (b, 0, 0)),
        ],
        out_specs=pl.BlockSpec((N, C, HB, W), lambda b, h: (b, 0, h, 0)),
        out_shape=jax.ShapeDtypeStruct((B, C, H, W), images.dtype),
        compiler_params=pltpu.CompilerParams(
            dimension_semantics=("parallel", "parallel"),
        ),
    )(img_t, rows, cols)
    return jnp.transpose(out_t, (0, 2, 3, 1))


# speedup vs baseline: 1.0188x; 1.0188x over previous
import numpy as np
import jax
import jax.numpy as jnp
from jax.experimental import pallas as pl
from jax.experimental.pallas import tpu as pltpu

_RATIO = 0.6
_RATE = 0.5


def _band_vectors(B, H, W):
    rng = np.random.default_rng(0)
    mask_size = int(max(H, W) * 2)
    lo = int(min(H * 0.5, W * 0.3))
    hi = int(max(H * 0.5, W * 0.3)) + 1
    top = (mask_size - H) // 2
    left = (mask_size - W) // 2
    rows = np.zeros((B, H), np.float32)
    cols = np.zeros((B, W), np.float32)
    for b in range(B):
        gridblock = int(rng.integers(lo, hi))
        if _RATIO == 1:
            length = int(rng.integers(1, gridblock + 1))
        else:
            length = int(min(max(int(gridblock * _RATIO + 0.5), 1), gridblock - 1))
        ind = []
        for _ in range(2):
            start_w = int(rng.integers(0, gridblock + 1))
            v = np.zeros(mask_size, np.float32)
            for i in range(mask_size // gridblock):
                s = gridblock * i + start_w
                e = min(s + length, mask_size)
                if e > s:
                    v[s:e] = 1.0
            ind.append(v)
        rate_cond = rng.random() < _RATE
        if rate_cond:
            rows[b] = ind[0][top:top + H]
            cols[b] = ind[1][left:left + W]
        else:
            rows[b] = 1.0
            cols[b] = 1.0
    return rows, cols


def _body(img_ref, row_ref, col_ref, out_ref):
    r = row_ref[:, 0, :]
    c = col_ref[:, 0, :]
    m = jnp.maximum(r[:, :, None], c[:, None, :])
    out_ref[...] = img_ref[...] * m[:, None]


def kernel(images):
    B, H, W, C = images.shape
    rows_np, cols_np = _band_vectors(B, H, W)
    rows = jnp.asarray(rows_np.reshape(B, 1, H))
    cols = jnp.asarray(cols_np.reshape(B, 1, W))
    img_t = jnp.transpose(images, (0, 3, 1, 2))
    N = 4
    out_t = pl.pallas_call(
        _body,
        grid=(B // N,),
        in_specs=[
            pl.BlockSpec((N, C, H, W), lambda b: (b, 0, 0, 0)),
            pl.BlockSpec((N, 1, H), lambda b: (b, 0, 0)),
            pl.BlockSpec((N, 1, W), lambda b: (b, 0, 0)),
        ],
        out_specs=pl.BlockSpec((N, C, H, W), lambda b: (b, 0, 0, 0)),
        out_shape=jax.ShapeDtypeStruct((B, C, H, W), images.dtype),
        compiler_params=pltpu.CompilerParams(
            dimension_semantics=("parallel",),
        ),
    )(img_t, rows, cols)
    return jnp.transpose(out_t, (0, 2, 3, 1))
